# baseline (speedup 1.0000x reference)
import functools

import jax
import jax.numpy as jnp
from jax.experimental import pallas as pl
from jax.experimental.pallas import tpu as pltpu


def _gap_hw_major_kernel(x_ref, o_ref, *, inv_hw):
    x = x_ref[...].astype(jnp.float32)
    s = (jnp.sum(x, axis=(0, 1)) * inv_hw).astype(o_ref.dtype)
    o_ref[...] = s.reshape(o_ref.shape)


@jax.jit
def _global_avg_pool_2d(x):
    N, C, H, W = x.shape
    inv_hw = 1.0 / float(H * W)

    xt = x.transpose(2, 3, 0, 1)

    nt = N
    itemsize = x.dtype.itemsize
    while nt > 8 and H * W * nt * C * itemsize > 16 * 1024 * 1024:
        nt //= 2

    if (nt * C) % 128 == 0 and N % nt == 0:
        out_shape = (N * C // 128, 128)
        out_spec = pl.BlockSpec((nt * C // 128, 128), lambda i: (i, 0))
    else:
        out_shape = (N, C)
        out_spec = pl.BlockSpec((nt, C), lambda i: (i, 0))

    out = pl.pallas_call(
        functools.partial(_gap_hw_major_kernel, inv_hw=inv_hw),
        out_shape=jax.ShapeDtypeStruct(out_shape, x.dtype),
        grid=(pl.cdiv(N, nt),),
        in_specs=[pl.BlockSpec((H, W, nt, C), lambda i: (0, 0, i, 0))],
        out_specs=out_spec,
        compiler_params=pltpu.CompilerParams(
            dimension_semantics=("arbitrary",),
            vmem_limit_bytes=48 * 1024 * 1024,
        ),
        cost_estimate=pl.CostEstimate(
            flops=N * C * H * W,
            transcendentals=0,
            bytes_accessed=(N * C * H * W + N * C) * itemsize,
        ),
    )(xt)

    return out.reshape(N, C, 1, 1)


def kernel(x):
    return _global_avg_pool_2d(x)


# speedup vs baseline: 1.0413x; 1.0413x over previous
import functools

import jax
import jax.numpy as jnp
from jax.experimental import pallas as pl
from jax.experimental.pallas import tpu as pltpu


def _gap_hw_major_kernel(x_ref, o_ref, *, inv_hw):
    x = x_ref[...].astype(jnp.float32)
    s = (jnp.sum(x, axis=(0, 1)) * inv_hw).astype(o_ref.dtype)
    o_ref[...] = s.reshape(o_ref.shape)


@jax.jit
def _global_avg_pool_2d(x):
    N, C, H, W = x.shape
    inv_hw = 1.0 / float(H * W)

    xt = x.transpose(2, 3, 0, 1)

    nt = N
    itemsize = x.dtype.itemsize
    while nt > 8 and H * W * nt * C * itemsize > 8 * 1024 * 1024:
        nt //= 2

    if (nt * C) % 128 == 0 and N % nt == 0:
        out_shape = (N * C // 128, 128)
        out_spec = pl.BlockSpec((nt * C // 128, 128), lambda i: (i, 0))
    else:
        out_shape = (N, C)
        out_spec = pl.BlockSpec((nt, C), lambda i: (i, 0))

    out = pl.pallas_call(
        functools.partial(_gap_hw_major_kernel, inv_hw=inv_hw),
        out_shape=jax.ShapeDtypeStruct(out_shape, x.dtype),
        grid=(pl.cdiv(N, nt),),
        in_specs=[pl.BlockSpec((H, W, nt, C), lambda i: (0, 0, i, 0))],
        out_specs=out_spec,
        compiler_params=pltpu.CompilerParams(
            dimension_semantics=("arbitrary",),
            vmem_limit_bytes=48 * 1024 * 1024,
        ),
        cost_estimate=pl.CostEstimate(
            flops=N * C * H * W,
            transcendentals=0,
            bytes_accessed=(N * C * H * W + N * C) * itemsize,
        ),
    )(xt)

    return out.reshape(N, C, 1, 1)


def kernel(x):
    return _global_avg_pool_2d(x)
